# baseline (speedup 1.0000x reference)
import functools

import jax
import jax.numpy as jnp
from jax import lax
from jax.experimental import pallas as pl
from jax.experimental.pallas import tpu as pltpu
from jax.experimental.pallas import tpu_sc as plsc

LANES = 16
N_CORES = 2
N_SUBCORES = 16


@functools.partial(jax.jit, static_argnums=())
def _run(user_ids, movie_ids, user_factors, movie_factors):
    B = user_ids.shape[0]
    F = user_factors.shape[1]
    NW = N_CORES * N_SUBCORES
    BPW = B // NW

    mesh = plsc.VectorSubcoreMesh(core_axis_name="c", subcore_axis_name="s")

    @functools.partial(
        pl.kernel,
        mesh=mesh,
        compiler_params=pltpu.CompilerParams(
            needs_layout_passes=False, use_tc_tiling_on_sc=False),
        out_type=jax.ShapeDtypeStruct((B,), jnp.float32),
        scratch_types=[
            pltpu.VMEM((BPW,), jnp.int32),
            pltpu.VMEM((BPW,), jnp.int32),
            pltpu.VMEM((BPW, F), jnp.float32),
            pltpu.VMEM((BPW, F), jnp.float32),
            pltpu.VMEM((BPW,), jnp.float32),
            pltpu.SemaphoreType.DMA,
            pltpu.SemaphoreType.DMA,
        ],
    )
    def sc_kernel(uids_hbm, mids_hbm, uf_hbm, mf_hbm, out_hbm,
                  uidx_v, midx_v, urows_v, mrows_v, out_v, sem_u, sem_m):
        wid = lax.axis_index("s") * N_CORES + lax.axis_index("c")
        base = wid * BPW

        pltpu.sync_copy(uids_hbm.at[pl.ds(base, BPW)], uidx_v)
        pltpu.sync_copy(mids_hbm.at[pl.ds(base, BPW)], midx_v)

        cp_u = pltpu.async_copy(uf_hbm.at[uidx_v], urows_v, sem_u)
        cp_m = pltpu.async_copy(mf_hbm.at[midx_v], mrows_v, sem_m)
        cp_u.wait()
        cp_m.wait()

        lane = lax.broadcasted_iota(jnp.int32, (LANES,), 0)

        def group(g, carry):
            rows = g * LANES + lane
            acc = jnp.zeros((LANES,), jnp.float32)
            for f in range(F):
                col = jnp.full((LANES,), f, jnp.int32)
                u = plsc.load_gather(urows_v, [rows, col])
                m = plsc.load_gather(mrows_v, [rows, col])
                acc = acc + u * m
            out_v[pl.ds(g * LANES, LANES)] = acc
            return carry

        lax.fori_loop(0, BPW // LANES, group, 0)

        pltpu.sync_copy(out_v, out_hbm.at[pl.ds(base, BPW)])

    return sc_kernel(user_ids, movie_ids, user_factors, movie_factors)


def kernel(user_ids, movie_ids, user_factors, movie_factors):
    out = _run(user_ids.astype(jnp.int32), movie_ids.astype(jnp.int32),
               user_factors, movie_factors)
    return out.reshape(-1, 1)
